# baseline (speedup 1.0000x reference)
import functools

import jax
import jax.numpy as jnp
from jax import lax
from jax.experimental import pallas as pl
from jax.experimental.pallas import tpu as pltpu
from jax.experimental.pallas import tpu_sc as plsc

LANES = 16
CHUNK = 64
NSLOT = 4


def _build(bn, num_rel, emb_dim):
    info = plsc.get_sparse_core_info()
    nc, ns = info.num_cores, info.num_subcores
    nw = nc * ns
    per_w = bn // nw
    nchunks = per_w // CHUNK
    assert per_w * nw == bn and nchunks * CHUNK == per_w
    assert nchunks % NSLOT == 0 and CHUNK % LANES == 0
    d_dim = emb_dim
    nsl = d_dim // LANES
    ngrp = CHUNK // LANES

    mesh = plsc.VectorSubcoreMesh(core_axis_name="c", subcore_axis_name="s")

    @functools.partial(
        pl.kernel,
        out_type=jax.ShapeDtypeStruct((bn,), jnp.float32),
        mesh=mesh,
        compiler_params=pltpu.CompilerParams(
            needs_layout_passes=False,
            use_tc_tiling_on_sc=False,
            disable_bounds_checks=True,
        ),
        scratch_types=[
            pltpu.VMEM((num_rel, d_dim), jnp.float32),
            pltpu.VMEM((2 * CHUNK,), jnp.int32),
            pltpu.VMEM((2 * CHUNK,), jnp.int32),
            pltpu.VMEM((2 * CHUNK,), jnp.int32),
            pltpu.VMEM((2 * CHUNK,), jnp.int32),
            pltpu.VMEM((CHUNK,), jnp.int32),
            pltpu.VMEM((CHUNK,), jnp.int32),
            pltpu.VMEM((CHUNK,), jnp.int32),
            pltpu.VMEM((CHUNK,), jnp.int32),
            pltpu.VMEM((2 * CHUNK, d_dim), jnp.float32),
            pltpu.VMEM((2 * CHUNK, d_dim), jnp.float32),
            pltpu.VMEM((2 * CHUNK, d_dim), jnp.float32),
            pltpu.VMEM((2 * CHUNK, d_dim), jnp.float32),
            pltpu.VMEM((per_w,), jnp.float32),
            pltpu.SemaphoreType.DMA,
            pltpu.SemaphoreType.DMA,
            pltpu.SemaphoreType.DMA,
            pltpu.SemaphoreType.DMA,
            pltpu.SemaphoreType.DMA,
            pltpu.SemaphoreType.DMA,
            pltpu.SemaphoreType.DMA,
            pltpu.SemaphoreType.DMA,
            pltpu.SemaphoreType.DMA,
        ],
    )
    def mdist_kernel(ent_flat_hbm, r_hbm, etab_hbm, rel_hbm, out_hbm,
                     rel_v, ei0, ei1, ei2, ei3, ri0, ri1, ri2, ri3,
                     er0, er1, er2, er3, out_v,
                     gsem0, gsem1, gsem2, gsem3,
                     isem0, isem1, isem2, isem3, rsem):
        ei = (ei0, ei1, ei2, ei3)
        ri = (ri0, ri1, ri2, ri3)
        er = (er0, er1, er2, er3)
        gsem = (gsem0, gsem1, gsem2, gsem3)
        isem = (isem0, isem1, isem2, isem3)

        wid = lax.axis_index("s") * nc + lax.axis_index("c")
        base = wid * per_w

        def issue_idx(chunk_id, slot):
            off = base + chunk_id * CHUNK
            pltpu.async_copy(ent_flat_hbm.at[pl.ds(2 * off, 2 * CHUNK)],
                             ei[slot], isem[slot])
            pltpu.async_copy(r_hbm.at[pl.ds(off, CHUNK)], ri[slot], isem[slot])

        def wait_idx(slot):
            pltpu.make_async_copy(ent_flat_hbm.at[pl.ds(0, 2 * CHUNK)],
                                  ei[slot], isem[slot]).wait()
            pltpu.make_async_copy(r_hbm.at[pl.ds(0, CHUNK)], ri[slot],
                                  isem[slot]).wait()

        def issue_gather(slot):
            pltpu.async_copy(etab_hbm.at[ei[slot]], er[slot], gsem[slot])

        def wait_gather(slot):
            pltpu.make_async_copy(etab_hbm.at[ei[slot]], er[slot],
                                  gsem[slot]).wait()

        pltpu.async_copy(rel_hbm, rel_v, rsem)
        issue_idx(0, 0)
        issue_idx(1, 1)
        issue_idx(2, 2)
        wait_idx(0)
        issue_gather(0)
        wait_idx(1)
        issue_gather(1)
        pltpu.make_async_copy(rel_hbm, rel_v, rsem).wait()

        lane_iota = lax.iota(jnp.int32, LANES)

        def compute(chunk_id, slot):
            rows = er[slot]
            obase = chunk_id * CHUNK

            @plsc.parallel_loop(0, ngrp)
            def _(g):
                ridxv = ri[slot][pl.ds(g * LANES, LANES)]
                ob = obase + g * LANES
                for j in range(LANES):
                    pj = g * LANES + j
                    rp = ridxv[j]
                    acc = None
                    for s in range(nsl):
                        sl = pl.ds(s * LANES, LANES)
                        v = (rows[2 * pj, sl] * rows[2 * pj + 1, sl]
                             * rel_v[rp, sl])
                        acc = v if acc is None else acc + v
                    tot = jnp.sum(acc)
                    plsc.store_scatter(
                        out_v,
                        [jnp.full((LANES,), ob + j, jnp.int32)],
                        jnp.full((LANES,), tot, jnp.float32),
                        mask=lane_iota == j,
                    )

        def body(i, carry):
            for b in range(NSLOT):
                chunk_id = i * NSLOT + b
                wait_gather(b)

                @pl.when(chunk_id + 3 < nchunks)
                def _():
                    issue_idx(chunk_id + 3, (b + 3) % NSLOT)

                @pl.when(chunk_id + 2 < nchunks)
                def _():
                    wait_idx((b + 2) % NSLOT)
                    issue_gather((b + 2) % NSLOT)

                compute(chunk_id, b)
            return carry

        lax.fori_loop(0, nchunks // NSLOT, body, 0)
        pltpu.sync_copy(out_v, out_hbm.at[pl.ds(base, per_w)])

    return mdist_kernel


@jax.jit
def kernel(r_idx, entities_idx, E_weight, R_weight):
    b, n = r_idx.shape
    bn = b * n
    ent_flat = entities_idx.reshape(2 * bn)
    rf = r_idx.reshape(bn)
    k = _build(bn, R_weight.shape[0], R_weight.shape[1])
    out = k(ent_flat, rf, E_weight, R_weight)
    return out.reshape(b, n)
